# baseline (speedup 1.0000x reference)
import jax
import jax.numpy as jnp
from jax import lax
from jax.experimental import pallas as pl
from jax.experimental.pallas import tpu as pltpu
from jax.experimental.pallas import tpu_sc as plsc

_B = 16384
_R = 128
_LANES = 16
_VPR = _R // _LANES

_NC = 2
_NS = 16
_NW = _NC * _NS
_SC_ROWS = 4096
_ROWS = _SC_ROWS // _NW
_CHUNK = 64
_NCHUNK = _ROWS // _CHUNK

_TC_ROWS = _B - _SC_ROWS
_BLK = 4096
_GPB = _BLK // 128


def _sc_body(u_hbm, d1_hbm, d2_hbm, v1_hbm, v2_hbm, out_hbm,
             su, sd1, sd2, p1b, p2b, vb1, vb2, ob,
             sem_in0, sem_in1, sem_out0, sem_out1):
    wid = lax.axis_index("c") * _NS + lax.axis_index("s")
    base = wid * _ROWS

    pltpu.sync_copy(u_hbm.at[pl.ds(base, _ROWS)], su)
    pltpu.sync_copy(d1_hbm.at[pl.ds(base, _ROWS)], sd1)
    pltpu.sync_copy(d2_hbm.at[pl.ds(base, _ROWS)], sd2)

    def pre(i, carry):
        sl = pl.ds(i * _LANES, _LANES)
        u = su[sl]
        z = jnp.zeros_like(u)
        p2 = jnp.minimum(sd2[sl], jnp.maximum(u, z))
        p1 = jnp.minimum(sd1[sl], jnp.maximum(u - p2, z))
        p2b[sl] = p2
        p1b[sl] = p1
        return carry

    lax.fori_loop(0, _ROWS // _LANES, pre, 0)

    in_sems = (sem_in0, sem_in1)
    out_sems = (sem_out0, sem_out1)

    def start_in(c):
        slot = c % 2
        rows = pl.ds(base + c * _CHUNK, _CHUNK)
        return (
            pltpu.async_copy(v1_hbm.at[rows, :], vb1.at[slot], in_sems[slot]),
            pltpu.async_copy(v2_hbm.at[rows, :], vb2.at[slot], in_sems[slot]),
        )

    in_flight = [start_in(0), start_in(1) if _NCHUNK > 1 else None]
    out_flight = [None, None]

    for c in range(_NCHUNK):
        slot = c % 2
        for cp in in_flight[slot]:
            cp.wait()
        if out_flight[slot] is not None:
            out_flight[slot].wait()

        coff = c * _CHUNK

        def row_group(g, carry):
            p1v = p1b[pl.ds(coff + g * _LANES, _LANES)]
            p2v = p2b[pl.ds(coff + g * _LANES, _LANES)]
            for k in range(_LANES):
                r = g * _LANES + k
                p1s = p1v[k]
                p2s = p2v[k]
                for j in range(_VPR):
                    sl = pl.ds(j * _LANES, _LANES)
                    acc = vb2[slot, r, sl] * p2s + vb1[slot, r, sl] * p1s
                    ob[slot, r, sl] = acc
            return carry

        lax.fori_loop(0, _CHUNK // _LANES, row_group, 0)

        out_flight[slot] = pltpu.async_copy(
            ob.at[slot], out_hbm.at[pl.ds(base + coff, _CHUNK), :],
            out_sems[slot])
        if c + 2 < _NCHUNK:
            in_flight[slot] = start_in(c + 2)

    for cp in out_flight:
        if cp is not None:
            cp.wait()


_sc_call = pl.kernel(
    _sc_body,
    out_type=jax.ShapeDtypeStruct((_SC_ROWS, _R), jnp.float32),
    mesh=plsc.VectorSubcoreMesh(core_axis_name="c", subcore_axis_name="s"),
    scratch_types=[
        pltpu.VMEM((_ROWS,), jnp.float32),
        pltpu.VMEM((_ROWS,), jnp.float32),
        pltpu.VMEM((_ROWS,), jnp.float32),
        pltpu.VMEM((_ROWS,), jnp.float32),
        pltpu.VMEM((_ROWS,), jnp.float32),
        pltpu.VMEM((2, _CHUNK, _R), jnp.float32),
        pltpu.VMEM((2, _CHUNK, _R), jnp.float32),
        pltpu.VMEM((2, _CHUNK, _R), jnp.float32),
        pltpu.SemaphoreType.DMA,
        pltpu.SemaphoreType.DMA,
        pltpu.SemaphoreType.DMA,
        pltpu.SemaphoreType.DMA,
    ],
)


def _tc_body(us_ref, d1s_ref, d2s_ref, v1_ref, v2_ref, o_ref):
    us = us_ref[:]
    p2s = jnp.minimum(d2s_ref[:], jnp.maximum(us, 0.0))
    p1s = jnp.minimum(d1s_ref[:], jnp.maximum(us - p2s, 0.0))
    p2t = jnp.transpose(p2s)
    p1t = jnp.transpose(p1s)
    for g in range(_GPB):
        rows = pl.ds(g * 128, 128)
        p2c = p2t[:, g:g + 1]
        p1c = p1t[:, g:g + 1]
        o_ref[rows, :] = v2_ref[rows, :] * p2c + v1_ref[rows, :] * p1c


_SC_BLKS = _SC_ROWS // _BLK
_SC_SGRP = _SC_ROWS // 128


def _tc_call(us, d1s, d2s, v1, v2):
    grid = (_TC_ROWS // _BLK,)
    scal_spec = pl.BlockSpec((_GPB, 128), lambda i: (i + _SC_BLKS, 0))
    vec_spec = pl.BlockSpec((_BLK, _R), lambda i: (i + _SC_BLKS, 0))
    return pl.pallas_call(
        _tc_body,
        grid=grid,
        in_specs=[scal_spec, scal_spec, scal_spec, vec_spec, vec_spec],
        out_specs=vec_spec,
        out_shape=jax.ShapeDtypeStruct((_B, _R), jnp.float32),
    )(us, d1s, d2s, v1, v2)


@jax.jit
def kernel(u, d1, d2, v1, v2):
    u = u.reshape(-1)
    d1 = d1.reshape(-1)
    d2 = d2.reshape(-1)
    sc_out = _sc_call(u, d1, d2, v1, v2)
    g = _B // 128
    tc_full = _tc_call(u.reshape(g, 128), d1.reshape(g, 128),
                       d2.reshape(g, 128), v1, v2)
    return lax.dynamic_update_slice(tc_full, sc_out, (0, 0))


# speedup vs baseline: 2.4598x; 2.4598x over previous
import jax
import jax.numpy as jnp
from jax import lax
from jax.experimental import pallas as pl

_BLK = 2048
_GPB = _BLK // 128


def _tc_body(us_ref, d1s_ref, d2s_ref, v1_ref, v2_ref, o_ref):
    us = us_ref[:]
    p2s = jnp.minimum(d2s_ref[:], jnp.maximum(us, 0.0))
    p1s = jnp.minimum(d1s_ref[:], jnp.maximum(us - p2s, 0.0))
    p2t = jnp.transpose(p2s)
    p1t = jnp.transpose(p1s)
    for g in range(_GPB):
        rows = pl.ds(g * 128, 128)
        p2c = p2t[:, g:g + 1]
        p1c = p1t[:, g:g + 1]
        o_ref[rows, :] = v2_ref[rows, :] * p2c + v1_ref[rows, :] * p1c


def tc_kernel(u, d1, d2, v1, v2):
    B, R = v1.shape
    G = B // 128
    us = u.reshape(G, 128)
    d1s = d1.reshape(G, 128)
    d2s = d2.reshape(G, 128)
    grid = (B // _BLK,)
    scal_spec = pl.BlockSpec((_GPB, 128), lambda i: (i, 0))
    vec_spec = pl.BlockSpec((_BLK, R), lambda i: (i, 0))
    return pl.pallas_call(
        _tc_body,
        grid=grid,
        in_specs=[scal_spec, scal_spec, scal_spec, vec_spec, vec_spec],
        out_specs=vec_spec,
        out_shape=jax.ShapeDtypeStruct((B, R), v1.dtype),
    )(us, d1s, d2s, v1, v2)


def kernel(u, d1, d2, v1, v2):
    return tc_kernel(u.reshape(-1), d1.reshape(-1), d2.reshape(-1), v1, v2)


# speedup vs baseline: 3.0842x; 1.2539x over previous
import jax
import jax.numpy as jnp
from jax import lax
from jax.experimental import pallas as pl

_BLK = 8192
_GPB = _BLK // 128


def _tc_body(us_ref, d1s_ref, d2s_ref, v1_ref, v2_ref, o_ref):
    us = us_ref[:]
    p2s = jnp.minimum(d2s_ref[:], jnp.maximum(us, 0.0))
    p1s = jnp.minimum(d1s_ref[:], jnp.maximum(us - p2s, 0.0))
    p2t = jnp.transpose(p2s)
    p1t = jnp.transpose(p1s)
    for g in range(_GPB):
        rows = pl.ds(g * 128, 128)
        p2c = p2t[:, g:g + 1]
        p1c = p1t[:, g:g + 1]
        o_ref[rows, :] = v2_ref[rows, :] * p2c + v1_ref[rows, :] * p1c


def tc_kernel(u, d1, d2, v1, v2):
    B, R = v1.shape
    G = B // 128
    us = u.reshape(G, 128)
    d1s = d1.reshape(G, 128)
    d2s = d2.reshape(G, 128)
    grid = (B // _BLK,)
    scal_spec = pl.BlockSpec((_GPB, 128), lambda i: (i, 0))
    vec_spec = pl.BlockSpec((_BLK, R), lambda i: (i, 0))
    return pl.pallas_call(
        _tc_body,
        grid=grid,
        in_specs=[scal_spec, scal_spec, scal_spec, vec_spec, vec_spec],
        out_specs=vec_spec,
        out_shape=jax.ShapeDtypeStruct((B, R), v1.dtype),
    )(us, d1s, d2s, v1, v2)


def kernel(u, d1, d2, v1, v2):
    return tc_kernel(u.reshape(-1), d1.reshape(-1), d2.reshape(-1), v1, v2)
